# baseline (speedup 1.0000x reference)
import functools

import numpy as np
import jax
import jax.numpy as jnp
from jax import lax
from jax.experimental import pallas as pl
from jax.experimental.pallas import tpu as pltpu
from jax.experimental.pallas import tpu_sc as plsc

_C, _T, _H, _W = 3, 32, 384, 384
_ALPHA = 4
_NSLOW = _T // _ALPHA
_IDX = tuple(int(v) for v in np.linspace(0.0, _T - 1, _NSLOW).astype(np.float32))

_NWORKERS = 32
_ROWS = 128
_NCHUNKS = _H // _ROWS
_JOBS = tuple(
    (c, s, k) for c in range(_C) for s in range(_NSLOW) for k in range(_NCHUNKS)
)


def _sc_gather_body(frames_hbm, slow_hbm, buf):
    cid = lax.axis_index("c")
    sid = lax.axis_index("s")
    wid = sid * 2 + cid

    for j, (ch, slot, k) in enumerate(_JOBS):
        @pl.when(wid == j % _NWORKERS)
        def _(ch=ch, slot=slot, k=k):
            t = _IDX[slot]
            pltpu.sync_copy(frames_hbm.at[ch, t, pl.ds(k * _ROWS, _ROWS)], buf)
            pltpu.sync_copy(buf, slow_hbm.at[ch, slot, pl.ds(k * _ROWS, _ROWS)])


_sc_gather = functools.partial(
    pl.kernel,
    mesh=plsc.VectorSubcoreMesh(core_axis_name="c", subcore_axis_name="s"),
    out_type=jax.ShapeDtypeStruct((_C, _NSLOW, _H, _W), jnp.float32),
    scratch_types=[pltpu.VMEM((_ROWS, _W), jnp.float32)],
)(_sc_gather_body)


def _tc_copy_body(in_ref, fast_ref):
    fast_ref[...] = in_ref[...]


def _tc_copy(frames):
    return pl.pallas_call(
        _tc_copy_body,
        grid=(_T,),
        in_specs=[pl.BlockSpec((_C, 1, _H, _W), lambda t: (0, t, 0, 0))],
        out_specs=pl.BlockSpec((_C, 1, _H, _W), lambda t: (0, t, 0, 0)),
        out_shape=jax.ShapeDtypeStruct((_C, _T, _H, _W), frames.dtype),
    )(frames)


def kernel(frames):
    slow = _sc_gather(frames)
    fast = _tc_copy(frames)
    return (slow, fast)
